# baseline (speedup 1.0000x reference)
import functools

import jax
import jax.numpy as jnp
from jax import lax
from jax.experimental import pallas as pl
from jax.experimental.pallas import tpu as pltpu
from jax.experimental.pallas import tpu_sc as plsc

N_ITEMS = 20000
EMB = 128
K_CORE = 3

NT = 16
CEDGE = 128
NSUB = 8
SCH = NSUB * CEDGE
ZR = 8
WR = 8


def _mesh():
    return plsc.VectorSubcoreMesh(core_axis_name="c", subcore_axis_name="s")


def _rsqrt16(x):
    i = lax.bitcast_convert_type(x, jnp.int32)
    i = jnp.int32(0x5F3759DF) - lax.shift_right_logical(i, 1)
    y = lax.bitcast_convert_type(i, jnp.float32)
    for _ in range(3):
        y = y * (1.5 - 0.5 * x * y * y)
    return y


def _inv_sqrt_deg(d16):
    return 1.0 / (d16 * _rsqrt16(d16) + 1e-8)


def _tile_quota(half):
    hi = -(-(half // NT) // 8) * 8
    rem = half - (NT - 1) * hi
    return hi, rem


def _zero_region(shared, zbuf, sid, off, half):
    hi, rem = _tile_quota(half)
    my0 = off + sid * hi
    my_rows = jnp.where(sid == NT - 1, rem, hi)

    def zb(i, carry):
        @pl.when(i * ZR < my_rows)
        def _():
            pltpu.sync_copy(zbuf, shared.at[pl.ds(my0 + i * ZR, ZR)])
        return carry
    lax.fori_loop(0, hi // ZR, zb, 0)


@functools.lru_cache(maxsize=None)
def _make_prep(n_a, n_b, e_pad):
    ha, hb = n_a // 2, n_b // 2
    maxh = max(ha, hb)
    maxq = max(_tile_quota(ha)[0] * NT, _tile_quota(hb)[0] * NT)
    maxh = max(maxh, maxq)
    nsc = e_pad // (NT * SCH)
    out_type = (jax.ShapeDtypeStruct((n_a, EMB), jnp.float32),
                jax.ShapeDtypeStruct((n_b, EMB), jnp.float32),
                jax.ShapeDtypeStruct((n_a, EMB), jnp.float32),
                jax.ShapeDtypeStruct((n_b, EMB), jnp.float32))
    scratch = [
        pltpu.VMEM_SHARED((maxh + 1, EMB), jnp.float32),
        pltpu.VMEM((SCH,), jnp.int32),
        pltpu.VMEM((NSUB, CEDGE), jnp.int32),
        pltpu.VMEM((CEDGE, EMB), jnp.float32),
        pltpu.VMEM((ZR, EMB), jnp.float32),
        pltpu.VMEM((WR, EMB), jnp.float32),
        pltpu.VMEM((WR, EMB), jnp.float32),
        pltpu.VMEM((WR, EMB), jnp.float32),
        pltpu.VMEM((WR, EMB), jnp.float32),
        pltpu.SemaphoreType.DMA,
    ]

    def body(rows_h, cols_h, a0_t, b0_t, p_h, q_h, abar_h, bbar_h,
             shared, didx, ldx, ones_b, zbuf, degb, pbuf, fb0, fb1, sem):
        sid = lax.axis_index("s")
        cid = lax.axis_index("c")
        zv = jnp.zeros((16,), jnp.float32)
        ov = jnp.full((16,), 1.0)
        for r in range(ZR):
            for c in range(EMB // 16):
                zbuf[r, pl.ds(c * 16, 16)] = zv
        for i in range(CEDGE):
            for c in range(EMB // 16):
                ones_b[i, pl.ds(c * 16, 16)] = ov

        def run_side(dst_h, half, base, feat_t, p_out, bar_out):
            hi, rem = _tile_quota(half)
            my0 = sid * hi
            my_rows = jnp.where(sid == NT - 1, rem, hi)

            _zero_region(shared, zbuf, sid, 0, half)
            plsc.subcore_barrier()

            def eb(ci, carry):
                e0 = (sid * nsc + ci) * SCH
                pltpu.sync_copy(dst_h.at[pl.ds(e0, SCH)], didx)
                for g in range(SCH // 16):
                    r, cc = g // 8, (g % 8) * 16
                    sl = pl.ds(g * 16, 16)
                    d = didx[sl] - base
                    ok = (d >= 0) & (d < half)
                    ldx[r, pl.ds(cc, 16)] = jnp.where(ok, d, half)
                for s in range(NSUB):
                    pltpu.sync_copy(ones_b, shared.at[ldx.at[s]], add=True)
                return carry
            lax.fori_loop(0, nsc, eb, 0)
            plsc.subcore_barrier()

            def wb(k, carry):
                @pl.when(k * WR < my_rows)
                def _():
                    s0 = my0 + k * WR
                    g0 = base + s0
                    pltpu.sync_copy(shared.at[pl.ds(s0, WR)], degb)
                    pltpu.sync_copy(feat_t.at[pl.ds(g0, WR)], fb0)
                    for r in range(WR):
                        pv = _inv_sqrt_deg(degb[r, pl.ds(0, 16)])
                        for c in range(EMB // 16):
                            sl = pl.ds(c * 16, 16)
                            pbuf[r, sl] = pv
                            fb1[r, sl] = fb0[r, sl] * pv
                    pltpu.sync_copy(pbuf, p_out.at[pl.ds(g0, WR)])
                    pltpu.sync_copy(fb1, bar_out.at[pl.ds(g0, WR)])
                return carry
            lax.fori_loop(0, hi // WR, wb, 0)
            plsc.subcore_barrier()

        run_side(rows_h, ha, cid * ha, a0_t, p_h, abar_h)
        run_side(cols_h, hb, cid * hb, b0_t, q_h, bbar_h)

    return pl.kernel(body, out_type=out_type, mesh=_mesh(),
                     scratch_types=scratch)


@functools.lru_cache(maxsize=None)
def _make_layer(n_a, n_b, e_pad, combine):
    ha, hb = n_a // 2, n_b // 2
    maxh = max(ha, hb)
    maxq = max(_tile_quota(ha)[0] * NT, _tile_quota(hb)[0] * NT)
    maxh = max(maxh, maxq)
    nsc = e_pad // (NT * SCH)

    if combine:
        out_type = (jax.ShapeDtypeStruct((n_a, EMB), jnp.float32),
                    jax.ShapeDtypeStruct((n_b, EMB), jnp.float32))
    else:
        out_type = (jax.ShapeDtypeStruct((n_a, EMB), jnp.float32),
                    jax.ShapeDtypeStruct((n_a, EMB), jnp.float32),
                    jax.ShapeDtypeStruct((n_b, EMB), jnp.float32),
                    jax.ShapeDtypeStruct((n_b, EMB), jnp.float32))
    scratch = [
        pltpu.VMEM_SHARED((maxh + 1, EMB), jnp.float32),
        pltpu.VMEM((SCH,), jnp.int32),
        pltpu.VMEM((NSUB, CEDGE), jnp.int32),
        pltpu.VMEM((SCH,), jnp.int32),
        pltpu.VMEM((NSUB, CEDGE), jnp.int32),
        pltpu.VMEM((CEDGE, EMB), jnp.float32),
        pltpu.VMEM((CEDGE, EMB), jnp.float32),
        pltpu.VMEM((ZR, EMB), jnp.float32),
        pltpu.VMEM((WR, EMB), jnp.float32),
        pltpu.VMEM((WR, EMB), jnp.float32),
        pltpu.VMEM((WR, EMB), jnp.float32),
        pltpu.VMEM((WR, EMB), jnp.float32),
        pltpu.SemaphoreType.DMA,
        pltpu.SemaphoreType.DMA,
        pltpu.SemaphoreType.DMA,
        pltpu.SemaphoreType.DMA,
    ]

    def body(*refs):
        if combine:
            (abar_t, bbar_t, rows_h, cols_h, p_t, q_t, a0_t, a1_t, b0_t, b1_t,
             out_a, out_b, shared, sidx, gidx, didx, lidx, rows0, rows1,
             zbuf, pbuf, buf0, buf1, buf2, sg0, sg1, ss0, ss1) = refs
        else:
            (abar_t, bbar_t, rows_h, cols_h, p_t, q_t,
             out_a, out_abar, out_b, out_bbar, shared, sidx, gidx, didx, lidx,
             rows0, rows1, zbuf, pbuf, buf0, buf1, buf2,
             sg0, sg1, ss0, ss1) = refs
            a0_t = a1_t = b0_t = b1_t = None
            out_a, out_b = out_a, out_b
        rows_b = (rows0, rows1)
        sg = (sg0, sg1)
        ss = (ss0, ss1)
        sid = lax.axis_index("s")
        cid = lax.axis_index("c")
        zv = jnp.zeros((16,), jnp.float32)
        for r in range(ZR):
            for c in range(EMB // 16):
                zbuf[r, pl.ds(c * 16, 16)] = zv

        def run_phase(dst_h, src_h, src_tbl, half, n_src, scale_t,
                      out_main, out_bar, prev0, prev1):
            base = cid * half
            hi, rem = _tile_quota(half)
            my0 = sid * hi
            my_rows = jnp.where(sid == NT - 1, rem, hi)

            _zero_region(shared, zbuf, sid, 0, half)
            plsc.subcore_barrier()

            def eb(ci, carry):
                e0 = (sid * nsc + ci) * SCH
                pltpu.sync_copy(src_h.at[pl.ds(e0, SCH)], sidx)
                pltpu.sync_copy(dst_h.at[pl.ds(e0, SCH)], didx)

                for g in range(SCH // 16):
                    r, cc = g // 8, (g % 8) * 16
                    sl = pl.ds(g * 16, 16)
                    slo = pl.ds(cc, 16)
                    gidx[r, slo] = jnp.minimum(sidx[sl], n_src - 1)
                    d = didx[sl] - base
                    ok = (d >= 0) & (d < half)
                    lidx[r, slo] = jnp.where(ok, d, half)

                for s in range(NSUB):
                    b = s % 2
                    pltpu.async_copy(
                        src_tbl.at[gidx.at[s]], rows_b[b], sg[b]).wait()
                    pltpu.sync_copy(rows_b[b], shared.at[lidx.at[s]],
                                    add=True)
                return carry
            lax.fori_loop(0, nsc, eb, 0)
            plsc.subcore_barrier()

            def wb(k, carry):
                @pl.when(k * WR < my_rows)
                def _():
                    s0 = my0 + k * WR
                    g0 = base + s0
                    pltpu.sync_copy(shared.at[pl.ds(s0, WR)], buf2)
                    pltpu.sync_copy(scale_t.at[pl.ds(g0, WR)], pbuf)
                    if combine:
                        pltpu.sync_copy(prev0.at[pl.ds(g0, WR)], buf0)
                        pltpu.sync_copy(prev1.at[pl.ds(g0, WR)], buf1)
                        for r in range(WR):
                            pv = pbuf[r, pl.ds(0, 16)]
                            for c in range(EMB // 16):
                                sl = pl.ds(c * 16, 16)
                                buf0[r, sl] = (buf0[r, sl] + buf1[r, sl]
                                               + buf2[r, sl] * pv) * (1.0 / 3.0)
                        pltpu.sync_copy(buf0, out_main.at[pl.ds(g0, WR)])
                    else:
                        for r in range(WR):
                            pv = pbuf[r, pl.ds(0, 16)]
                            for c in range(EMB // 16):
                                sl = pl.ds(c * 16, 16)
                                t = buf2[r, sl] * pv
                                buf0[r, sl] = t
                                buf1[r, sl] = t * pv
                        pltpu.sync_copy(buf0, out_main.at[pl.ds(g0, WR)])
                        pltpu.sync_copy(buf1, out_bar.at[pl.ds(g0, WR)])
                return carry
            lax.fori_loop(0, hi // WR, wb, 0)
            plsc.subcore_barrier()

        if combine:
            run_phase(rows_h, cols_h, bbar_t, ha, n_b, p_t,
                      out_a, None, a0_t, a1_t)
            run_phase(cols_h, rows_h, abar_t, hb, n_a, q_t,
                      out_b, None, b0_t, b1_t)
        else:
            run_phase(rows_h, cols_h, bbar_t, ha, n_b, p_t,
                      out_a, out_abar, None, None)
            run_phase(cols_h, rows_h, abar_t, hb, n_a, q_t,
                      out_b, out_bbar, None, None)

    return pl.kernel(body, out_type=out_type, mesh=_mesh(),
                     scratch_types=scratch)


def _pad_edges(rows, cols, n_a, n_b):
    e = rows.shape[0]
    e_pad = -(-e // (NT * SCH)) * (NT * SCH)
    pad = e_pad - e
    rows_p = jnp.concatenate([rows.astype(jnp.int32),
                              jnp.full((pad,), n_a, jnp.int32)])
    cols_p = jnp.concatenate([cols.astype(jnp.int32),
                              jnp.full((pad,), n_b, jnp.int32)])
    return rows_p, cols_p, e_pad


def _propagate_sc(rows, cols, feat_a, feat_b):
    n_a, n_b = feat_a.shape[0], feat_b.shape[0]
    rows_p, cols_p, e_pad = _pad_edges(rows, cols, n_a, n_b)
    prep = _make_prep(n_a, n_b, e_pad)
    p_t, q_t, abar0, bbar0 = prep(rows_p, cols_p, feat_a, feat_b)
    l1 = _make_layer(n_a, n_b, e_pad, False)
    a1, abar1, b1, bbar1 = l1(abar0, bbar0, rows_p, cols_p, p_t, q_t)
    l2 = _make_layer(n_a, n_b, e_pad, True)
    return l2(abar1, bbar1, rows_p, cols_p, p_t, q_t,
              feat_a, a1, feat_b, b1)


def _propagate_xla(rows, cols, w, feat_a, feat_b):
    a, b = feat_a, feat_b
    acc_a, acc_b = feat_a, feat_b
    for _ in range(2):
        na = jnp.zeros_like(a).at[rows].add(w[:, None] * b[cols])
        nb = jnp.zeros_like(b).at[cols].add(w[:, None] * a[rows])
        a, b = na, nb
        acc_a = acc_a + a
        acc_b = acc_b + b
    return acc_a / 3.0, acc_b / 3.0


def kernel(users, bundles, users_feature, bundles_feature, items_feature, ub_rows, ub_cols, ub_w, ui_rows, ui_cols, ui_w, bi_rows, bi_cols, bi_w, bundle_items, w_core1, b_core1, w_core2, b_core2, w_syn1, b_syn1, w_syn2, b_syn2):
    UB_u, UB_b = _propagate_sc(ub_rows, ub_cols, users_feature, bundles_feature)
    UI_u, UI_i = _propagate_sc(ui_rows, ui_cols, users_feature, items_feature)
    BI_b, BI_i = _propagate_sc(bi_rows, bi_cols, bundles_feature, items_feature)
    num_candidates = bundles.shape[1]
    users_expanded = jnp.repeat(users, num_candidates)
    bundles_flat = bundles.reshape(-1)
    bbi = bundle_items[bundles_flat]
    mask = bbi != N_ITEMS
    UI_i_pad = jnp.concatenate([UI_i, jnp.zeros((1, EMB), jnp.float32)], axis=0)
    BI_i_pad = jnp.concatenate([BI_i, jnp.zeros((1, EMB), jnp.float32)], axis=0)
    items_ui = UI_i_pad[bbi]
    items_bi = BI_i_pad[bbi]
    u_ui = UI_u[users_expanded]
    b_bi = BI_b[bundles_flat]
    r_ui = jnp.sum(u_ui[:, None, :] * items_ui, axis=2)
    r_bi = jnp.sum(b_bi[:, None, :] * items_bi, axis=2)
    mlp_in = jnp.stack([r_ui, r_bi], axis=2)
    h = jax.nn.relu(mlp_in @ w_core1.T + b_core1)
    core_logits = (h @ w_core2.T + b_core2)[..., 0]
    core_logits = jnp.where(mask, core_logits, -jnp.inf)
    pi = jax.nn.softmax(core_logits, axis=1)
    k = min(K_CORE, bbi.shape[1])
    topk_vals, topk_idx = jax.lax.top_k(pi, k)
    topk_pi = topk_vals / (jnp.sum(topk_vals, axis=1, keepdims=True) + 1e-10)
    core_items = jnp.take_along_axis(items_ui, topk_idx[:, :, None], axis=1)
    h_core = jnp.sum(core_items * topk_pi[:, :, None], axis=1)
    is_core = jnp.zeros(pi.shape, bool).at[jnp.arange(pi.shape[0])[:, None], topk_idx].set(True)
    is_fringe = mask & (~is_core)
    fringe_sum = jnp.sum(items_ui * is_fringe[:, :, None].astype(jnp.float32), axis=1)
    fringe_count = jnp.maximum(jnp.sum(is_fringe, axis=1, keepdims=True).astype(jnp.float32), 1.0)
    h_fringe = fringe_sum / fringe_count
    syn_h = jax.nn.relu(jnp.concatenate([h_core, h_fringe], axis=1) @ w_syn1.T + b_syn1)
    h_syn = syn_h @ w_syn2.T + b_syn2
    synergy = jnp.sum(u_ui * h_syn, axis=1)
    main = jnp.sum(UB_u[users_expanded] * UB_b[bundles_flat], axis=1)
    return (main + synergy).reshape(bundles.shape)


# speedup vs baseline: 1.8653x; 1.8653x over previous
import functools

import jax
import jax.numpy as jnp
from jax import lax
from jax.experimental import pallas as pl
from jax.experimental.pallas import tpu as pltpu
from jax.experimental.pallas import tpu_sc as plsc

N_USERS = 20000
N_BUNDLES = 8000
N_ITEMS = 20000
EMB = 128
LAYERS = 2
K_CORE = 3

NT = 16
NC = 2
CEDGE = 128
ZR = 8
WR = 8


@functools.lru_cache(maxsize=None)
def _make_prop(n_a, n_b, e_pad, combine):
    ha, hb = n_a // 2, n_b // 2
    maxh = max(ha, hb)
    maxq = max(-(-(ha // NT) // 8) * 8 * NT, -(-(hb // NT) // 8) * 8 * NT)
    maxh = max(maxh, maxq)
    ch = e_pad // (NT * CEDGE)
    mesh = plsc.VectorSubcoreMesh(core_axis_name="c", subcore_axis_name="s")

    out_type = (jax.ShapeDtypeStruct((n_a, EMB), jnp.float32),
                jax.ShapeDtypeStruct((n_b, EMB), jnp.float32))
    scratch = [
        pltpu.VMEM_SHARED((maxh + 1, EMB), jnp.float32),
        pltpu.VMEM((CEDGE,), jnp.int32),
        pltpu.VMEM((CEDGE,), jnp.int32),
        pltpu.VMEM((CEDGE,), jnp.int32),
        pltpu.VMEM((CEDGE,), jnp.int32),
        pltpu.VMEM((CEDGE,), jnp.float32),
        pltpu.VMEM((CEDGE, EMB), jnp.float32),
        pltpu.VMEM((ZR, EMB), jnp.float32),
        pltpu.VMEM((WR, EMB), jnp.float32),
        pltpu.VMEM((WR, EMB), jnp.float32),
        pltpu.VMEM((WR, EMB), jnp.float32),
        pltpu.SemaphoreType.DMA,
    ]

    def body(*refs):
        if combine:
            (a_t, b_t, rows_h, cols_h, w_h, a0_t, b0_t, out_a, out_b,
             shared, sidx, gidx, didx, lidx, wv, rows, zbuf, buf0, buf1, buf2,
             sem) = refs
        else:
            (a_t, b_t, rows_h, cols_h, w_h, out_a, out_b,
             shared, sidx, gidx, didx, lidx, wv, rows, zbuf, buf0, buf1, buf2,
             sem) = refs
            a0_t = b0_t = None
        sid = lax.axis_index("s")
        cid = lax.axis_index("c")

        zv = jnp.zeros((16,), jnp.float32)
        for r in range(ZR):
            for c in range(EMB // 16):
                zbuf[r, pl.ds(c * 16, 16)] = zv

        def run_phase(dst_h, src_h, src_tbl, half, n_src, out_h, prev0, prev1):
            base = cid * half
            share_hi = -(-(half // NT) // 8) * 8
            rem = half - (NT - 1) * share_hi
            my0 = sid * share_hi
            my_rows = jnp.where(sid == NT - 1, rem, share_hi)

            def zb(i, carry):
                @pl.when(i * ZR < my_rows)
                def _():
                    pltpu.sync_copy(zbuf, shared.at[pl.ds(my0 + i * ZR, ZR)])
                return carry
            lax.fori_loop(0, share_hi // ZR, zb, 0)
            plsc.subcore_barrier()

            def eb(ci, carry):
                e0 = (sid * ch + ci) * CEDGE
                pltpu.sync_copy(src_h.at[pl.ds(e0, CEDGE)], sidx)
                pltpu.sync_copy(dst_h.at[pl.ds(e0, CEDGE)], didx)
                pltpu.sync_copy(w_h.at[pl.ds(e0, CEDGE)], wv)
                for q in range(CEDGE // 16):
                    sl = pl.ds(q * 16, 16)
                    gidx[sl] = jnp.minimum(sidx[sl], n_src - 1)
                    d = didx[sl] - base
                    ok = (d >= 0) & (d < half)
                    lidx[sl] = jnp.where(ok, d, half)
                pltpu.async_copy(src_tbl.at[gidx], rows, sem).wait()
                for q in range(CEDGE // 16):
                    w16 = wv[pl.ds(q * 16, 16)]
                    for jj in range(16):
                        j = q * 16 + jj
                        wj = jnp.full((16,), w16[jj])
                        for c in range(EMB // 16):
                            sl = pl.ds(c * 16, 16)
                            rows[j, sl] = rows[j, sl] * wj
                pltpu.sync_copy(rows, shared.at[lidx], add=True)
                return carry
            lax.fori_loop(0, ch, eb, 0)
            plsc.subcore_barrier()

            if not combine:
                @pl.when(sid < NT - 1)
                def _():
                    pltpu.sync_copy(shared.at[pl.ds(my0, share_hi)],
                                    out_h.at[pl.ds(base + my0, share_hi)])

                @pl.when(sid == NT - 1)
                def _():
                    pltpu.sync_copy(shared.at[pl.ds(my0, rem)],
                                    out_h.at[pl.ds(base + my0, rem)])
            else:
                def wb(k, carry):
                    s0 = my0 + k * WR
                    g0 = base + s0

                    @pl.when(k * WR < my_rows)
                    def _():
                        pltpu.sync_copy(prev0.at[pl.ds(g0, WR)], buf0)
                        pltpu.sync_copy(prev1.at[pl.ds(g0, WR)], buf1)
                        pltpu.sync_copy(shared.at[pl.ds(s0, WR)], buf2)
                        for r in range(WR):
                            for c in range(EMB // 16):
                                sl = pl.ds(c * 16, 16)
                                buf0[r, sl] = (buf0[r, sl] + buf1[r, sl]
                                               + buf2[r, sl]) * (1.0 / 3.0)
                        pltpu.sync_copy(buf0, out_h.at[pl.ds(g0, WR)])
                    return carry
                lax.fori_loop(0, share_hi // WR, wb, 0)
            plsc.subcore_barrier()

        run_phase(rows_h, cols_h, b_t, ha, n_b, out_a, a0_t, a_t)
        run_phase(cols_h, rows_h, a_t, hb, n_a, out_b, b0_t, b_t)

    return pl.kernel(body, out_type=out_type, mesh=mesh, scratch_types=scratch)


def _pad_edges(rows, cols, w, n_a, n_b):
    e = rows.shape[0]
    e_pad = -(-e // (NT * CEDGE)) * (NT * CEDGE)
    pad = e_pad - e
    rows_p = jnp.concatenate([rows.astype(jnp.int32),
                              jnp.full((pad,), n_a, jnp.int32)])
    cols_p = jnp.concatenate([cols.astype(jnp.int32),
                              jnp.full((pad,), n_b, jnp.int32)])
    w_p = jnp.concatenate([w, jnp.zeros((pad,), jnp.float32)])
    return rows_p, cols_p, w_p, e_pad


def _propagate_sc(rows, cols, w, feat_a, feat_b):
    n_a, n_b = feat_a.shape[0], feat_b.shape[0]
    rows_p, cols_p, w_p, e_pad = _pad_edges(rows, cols, w, n_a, n_b)
    l1 = _make_prop(n_a, n_b, e_pad, False)
    a1, b1 = l1(feat_a, feat_b, rows_p, cols_p, w_p)
    l2 = _make_prop(n_a, n_b, e_pad, True)
    return l2(a1, b1, rows_p, cols_p, w_p, feat_a, feat_b)


def kernel(users, bundles, users_feature, bundles_feature, items_feature, ub_rows, ub_cols, ub_w, ui_rows, ui_cols, ui_w, bi_rows, bi_cols, bi_w, bundle_items, w_core1, b_core1, w_core2, b_core2, w_syn1, b_syn1, w_syn2, b_syn2):
    UB_u, UB_b = _propagate_sc(ub_rows, ub_cols, ub_w, users_feature, bundles_feature)
    UI_u, UI_i = _propagate_sc(ui_rows, ui_cols, ui_w, users_feature, items_feature)
    BI_b, BI_i = _propagate_sc(bi_rows, bi_cols, bi_w, bundles_feature, items_feature)
    num_candidates = bundles.shape[1]
    users_expanded = jnp.repeat(users, num_candidates)
    bundles_flat = bundles.reshape(-1)
    bbi = bundle_items[bundles_flat]
    mask = bbi != N_ITEMS
    UI_i_pad = jnp.concatenate([UI_i, jnp.zeros((1, EMB), jnp.float32)], axis=0)
    BI_i_pad = jnp.concatenate([BI_i, jnp.zeros((1, EMB), jnp.float32)], axis=0)
    items_ui = UI_i_pad[bbi]
    items_bi = BI_i_pad[bbi]
    u_ui = UI_u[users_expanded]
    b_bi = BI_b[bundles_flat]
    r_ui = jnp.sum(u_ui[:, None, :] * items_ui, axis=2)
    r_bi = jnp.sum(b_bi[:, None, :] * items_bi, axis=2)
    mlp_in = jnp.stack([r_ui, r_bi], axis=2)
    h = jax.nn.relu(mlp_in @ w_core1.T + b_core1)
    core_logits = (h @ w_core2.T + b_core2)[..., 0]
    core_logits = jnp.where(mask, core_logits, -jnp.inf)
    pi = jax.nn.softmax(core_logits, axis=1)
    k = min(K_CORE, bbi.shape[1])
    topk_vals, topk_idx = jax.lax.top_k(pi, k)
    topk_pi = topk_vals / (jnp.sum(topk_vals, axis=1, keepdims=True) + 1e-10)
    core_items = jnp.take_along_axis(items_ui, topk_idx[:, :, None], axis=1)
    h_core = jnp.sum(core_items * topk_pi[:, :, None], axis=1)
    is_core = jnp.zeros(pi.shape, bool).at[jnp.arange(pi.shape[0])[:, None], topk_idx].set(True)
    is_fringe = mask & (~is_core)
    fringe_sum = jnp.sum(items_ui * is_fringe[:, :, None].astype(jnp.float32), axis=1)
    fringe_count = jnp.maximum(jnp.sum(is_fringe, axis=1, keepdims=True).astype(jnp.float32), 1.0)
    h_fringe = fringe_sum / fringe_count
    syn_h = jax.nn.relu(jnp.concatenate([h_core, h_fringe], axis=1) @ w_syn1.T + b_syn1)
    h_syn = syn_h @ w_syn2.T + b_syn2
    synergy = jnp.sum(u_ui * h_syn, axis=1)
    main = jnp.sum(UB_u[users_expanded] * UB_b[bundles_flat], axis=1)
    return (main + synergy).reshape(bundles.shape)
